# initial kernel scaffold (unmeasured)
import functools

import jax
import jax.numpy as jnp
from jax import lax
from jax.experimental import pallas as pl
from jax.experimental.pallas import tpu as pltpu

N_DEV = 32
E_PER = 4
C = 128
D = 512
H = 1024
N_TOK = 2048


def _moe_body(xg_ref, slot_ref, w_ref, out_ref, send_sems, recv_sems):
    my = lax.axis_index("i")
    left = lax.rem(my + N_DEV - 1, N_DEV)
    right = lax.rem(my + 1, N_DEV)

    barrier = pltpu.get_barrier_semaphore()
    for nbr in (left, right):
        pl.semaphore_signal(barrier, inc=1, device_id=(nbr,),
                            device_id_type=pl.DeviceIdType.MESH)
    pl.semaphore_wait(barrier, 2)

    acc = jnp.zeros((C, H), jnp.float32)
    for k in range(E_PER):
        mask = (slot_ref[:, :] == k).astype(jnp.float32)
        acc = acc + jnp.dot(xg_ref[:, :] * mask, w_ref[k],
                            preferred_element_type=jnp.float32)
    out_ref[pl.ds(my, 1)] = acc[None]

    for h in range(N_DEV - 1):
        origin = lax.rem(my - h + N_DEV, N_DEV)
        rdma = pltpu.make_async_remote_copy(
            src_ref=out_ref.at[origin],
            dst_ref=out_ref.at[origin],
            send_sem=send_sems.at[h],
            recv_sem=recv_sems.at[h],
            device_id=(right,),
            device_id_type=pl.DeviceIdType.MESH,
        )
        rdma.start()
        rdma.wait()

    @functools.partial(pl.run_scoped, sem=pltpu.SemaphoreType.REGULAR)
    def _(sem):
        for nbr in (left, right):
            pl.semaphore_signal(sem, inc=1, device_id=(nbr,),
                                device_id_type=pl.DeviceIdType.MESH)
        pl.semaphore_wait(sem, 2)


def kernel(x, router_W, route_idx, expert_W):
    del router_W
    my = lax.axis_index("i")

    ridx = route_idx[:, 0].astype(jnp.int32)
    dev = ridx // E_PER
    perm = jnp.argsort(ridx)
    counts = jnp.zeros((N_DEV,), jnp.int32).at[dev].add(1)
    starts = jnp.concatenate(
        [jnp.zeros((1,), jnp.int32), jnp.cumsum(counts)[:-1].astype(jnp.int32)])
    rank = jnp.zeros((N_TOK,), jnp.int32).at[perm].set(
        jnp.arange(N_TOK, dtype=jnp.int32))

    perm_pad = jnp.concatenate(
        [perm.astype(jnp.int32), jnp.zeros((C,), jnp.int32)])
    start_my = starts[my]
    idx_my = lax.dynamic_slice(perm_pad, (start_my,), (C,))
    e_my = ridx[idx_my] - E_PER * my
    in_range = jnp.arange(C, dtype=jnp.int32) < counts[my]
    slot = jnp.where((e_my >= 0) & (e_my < E_PER) & in_range,
                     e_my, -1).astype(jnp.int32)
    xg = x[idx_my]

    blocks = pl.pallas_call(
        _moe_body,
        out_shape=jax.ShapeDtypeStruct((N_DEV, C, H), jnp.float32),
        in_specs=[
            pl.BlockSpec(memory_space=pltpu.VMEM),
            pl.BlockSpec(memory_space=pltpu.VMEM),
            pl.BlockSpec(memory_space=pltpu.VMEM),
        ],
        out_specs=pl.BlockSpec(memory_space=pltpu.VMEM),
        scratch_shapes=[
            pltpu.SemaphoreType.DMA((N_DEV - 1,)),
            pltpu.SemaphoreType.DMA((N_DEV - 1,)),
        ],
        compiler_params=pltpu.CompilerParams(collective_id=0),
    )(xg, slot[:, None], expert_W)

    pos = rank - starts[dev]
    flat = dev * C + jnp.clip(pos, 0, C - 1)
    return blocks.reshape(N_DEV * C, H)[flat]


# baseline (device time: 314553 ns/iter reference)
import functools

import jax
import jax.numpy as jnp
from jax import lax
from jax.experimental import pallas as pl
from jax.experimental.pallas import tpu as pltpu

N_DEV = 32
E_PER = 4
N_TOK = 2048
D = 512
H = 1024
ROWS = N_TOK // N_DEV


def _moe_body(x_ref, ridx_ref, w_ref, out_ref, stage_ref,
              rs_send, rs_recv, ag_send, ag_recv):
    my = lax.axis_index("i")
    left = lax.rem(my + N_DEV - 1, N_DEV)
    right = lax.rem(my + 1, N_DEV)

    barrier = pltpu.get_barrier_semaphore()
    for nbr in (left, right):
        pl.semaphore_signal(barrier, inc=1, device_id=(nbr,),
                            device_id_type=pl.DeviceIdType.MESH)
    pl.semaphore_wait(barrier, 2)

    acc = jnp.zeros((N_TOK, H), jnp.float32)
    for k in range(E_PER):
        mask = (ridx_ref[:, :] == E_PER * my + k).astype(jnp.float32)
        acc = acc + jnp.dot(x_ref[:, :] * mask, w_ref[k],
                            preferred_element_type=jnp.float32)
    out_ref[:, :] = acc

    for h in range(N_DEV - 1):
        s = lax.rem(my - h + 2 * N_DEV, N_DEV)
        rdma = pltpu.make_async_remote_copy(
            src_ref=out_ref.at[pl.ds(s * ROWS, ROWS)],
            dst_ref=stage_ref.at[h],
            send_sem=rs_send.at[h],
            recv_sem=rs_recv.at[h],
            device_id=(right,),
            device_id_type=pl.DeviceIdType.MESH,
        )
        rdma.start()
        rdma.wait()
        r = lax.rem(my - h - 1 + 2 * N_DEV, N_DEV)
        sl = pl.ds(r * ROWS, ROWS)
        out_ref[sl, :] = out_ref[sl, :] + stage_ref[h]

    for h in range(N_DEV - 1):
        s = lax.rem(my + 1 - h + 2 * N_DEV, N_DEV)
        sl = pl.ds(s * ROWS, ROWS)
        rdma = pltpu.make_async_remote_copy(
            src_ref=out_ref.at[sl],
            dst_ref=out_ref.at[sl],
            send_sem=ag_send.at[h],
            recv_sem=ag_recv.at[h],
            device_id=(right,),
            device_id_type=pl.DeviceIdType.MESH,
        )
        rdma.start()
        rdma.wait()

    @functools.partial(pl.run_scoped, sem=pltpu.SemaphoreType.REGULAR)
    def _(sem):
        for nbr in (left, right):
            pl.semaphore_signal(sem, inc=1, device_id=(nbr,),
                                device_id_type=pl.DeviceIdType.MESH)
        pl.semaphore_wait(sem, 2)


def kernel(x, router_W, route_idx, expert_W):
    del router_W
    return pl.pallas_call(
        _moe_body,
        out_shape=jax.ShapeDtypeStruct((N_TOK, H), jnp.float32),
        in_specs=[
            pl.BlockSpec(memory_space=pltpu.VMEM),
            pl.BlockSpec(memory_space=pltpu.VMEM),
            pl.BlockSpec(memory_space=pltpu.VMEM),
        ],
        out_specs=pl.BlockSpec(memory_space=pltpu.VMEM),
        scratch_shapes=[
            pltpu.VMEM((N_DEV - 1, ROWS, H), jnp.float32),
            pltpu.SemaphoreType.DMA((N_DEV - 1,)),
            pltpu.SemaphoreType.DMA((N_DEV - 1,)),
            pltpu.SemaphoreType.DMA((N_DEV - 1,)),
            pltpu.SemaphoreType.DMA((N_DEV - 1,)),
        ],
        compiler_params=pltpu.CompilerParams(collective_id=0),
    )(x, route_idx.astype(jnp.int32), expert_W)
